# baseline (speedup 1.0000x reference)
import jax
import jax.numpy as jnp
from jax import lax
from jax.experimental import pallas as pl
from jax.experimental.pallas import tpu as pltpu
from jax.experimental.pallas import tpu_sc as plsc

N_RESP = 100000
D = 32
S = 200
B = 4096
NC = 2
NS = 16
NW = NC * NS
BB = B // NW
GROUP = 4
NGROUP = S // GROUP
NPAIR = NGROUP // 2
TILE = 8 * BB
D8STRIDE = NW * TILE
SROW = (D // 8) * D8STRIDE
OUT_WORDS = S * SROW
PITCH = BB + 1
OROWS = GROUP * 4 * 8


def _body(resp4_hbm, table_hbm, pos_hbm, out_hbm,
          idx_v, rowsA, rowsB, outA, outB, pos_v, gA, gB, sA, sB):
    wid = lax.axis_index("s") * NC + lax.axis_index("c")
    pltpu.sync_copy(pos_hbm, pos_v)
    pltpu.sync_copy(resp4_hbm.at[:, wid], idx_v)

    lane = lax.iota(jnp.int32, 16)
    rows_lo = lane
    rows_hi = lane + 16
    cols = [jnp.full((16,), u, jnp.int32) for u in range(16)]

    def fire_g(g, rows, sem):
        for sl in range(GROUP):
            s = g * GROUP + sl
            pltpu.async_copy(
                table_hbm.at[idx_v.at[s >> 3, s & 7]],
                rows.at[pl.ds(sl * BB, BB)],
                sem,
            )

    def drain_g(rows, sem):
        for sl in range(GROUP):
            pltpu.make_async_copy(
                table_hbm.at[idx_v.at[0, 0]], rows.at[pl.ds(sl * BB, BB)], sem
            ).wait()

    def fire_st(s0, outbuf, sem):
        for sl in range(GROUP):
            for d8 in range(4):
                pltpu.async_copy(
                    outbuf.at[pl.ds((sl * 4 + d8) * 8, 8), pl.ds(0, BB)],
                    out_hbm.at[pl.ds((s0 + sl) * (SROW // BB)
                                     + d8 * (D8STRIDE // BB) + wid * 8, 8), :],
                    sem,
                )

    def drain_st(outbuf, sem):
        pltpu.make_async_copy(
            outbuf.at[pl.ds(0, OROWS), pl.ds(0, BB)],
            out_hbm.at[pl.ds(0, OROWS), :], sem
        ).wait()

    def compute(s0, rows, outbuf):
        for sl in range(GROUP):
            s = s0 + sl
            p0 = pos_v[s, pl.ds(0, 16)]
            p1 = pos_v[s, pl.ds(16, 16)]

            @plsc.parallel_loop(0, BB // 8, unroll=2)
            def bi_body(k, sl=sl, p0=p0, p1=p1):
                dst = outbuf.at[pl.ds(sl * 32, 32)]
                lo = [rows[sl * BB + k * 8 + u, pl.ds(0, 16)] + p0
                      for u in range(8)]
                hi = [rows[sl * BB + k * 8 + u, pl.ds(16, 16)] + p1
                      for u in range(8)]
                cks = [cols[u] + k * 8 for u in range(8)]
                for u in range(8):
                    plsc.store_scatter(dst, [rows_lo, cks[u]], lo[u])
                    plsc.store_scatter(dst, [rows_hi, cks[u]], hi[u])

    fire_g(0, rowsA, gA)

    def pair_body(gp, carry):
        gA_id = 2 * gp
        s0 = gA_id * GROUP

        pl.when(gp > 0)(lambda: drain_st(outA, sA))
        fire_g(gA_id + 1, rowsB, gB)
        drain_g(rowsA, gA)
        compute(s0, rowsA, outA)
        fire_st(s0, outA, sA)

        pl.when(gp > 0)(lambda: drain_st(outB, sB))

        def _fire_next():
            fire_g(gA_id + 2, rowsA, gA)

        pl.when(gp < NPAIR - 1)(_fire_next)
        drain_g(rowsB, gB)
        compute(s0 + GROUP, rowsB, outB)
        fire_st(s0 + GROUP, outB, sB)
        return carry

    lax.fori_loop(0, NPAIR, pair_body, 0)
    drain_st(outA, sA)
    drain_st(outB, sB)


_sc_kernel = pl.kernel(
    _body,
    out_type=jax.ShapeDtypeStruct((OUT_WORDS // BB, BB), jnp.float32),
    mesh=plsc.VectorSubcoreMesh(
        core_axis_name="c", subcore_axis_name="s", num_cores=NC, num_subcores=NS
    ),
    scratch_types=[
        pltpu.VMEM((S // 8, 8, BB), jnp.int32),
        pltpu.VMEM((GROUP * BB, D), jnp.float32),
        pltpu.VMEM((GROUP * BB, D), jnp.float32),
        pltpu.VMEM((OROWS, PITCH), jnp.float32),
        pltpu.VMEM((OROWS, PITCH), jnp.float32),
        pltpu.VMEM((S, D), jnp.float32),
        pltpu.SemaphoreType.DMA,
        pltpu.SemaphoreType.DMA,
        pltpu.SemaphoreType.DMA,
        pltpu.SemaphoreType.DMA,
    ],
    compiler_params=pltpu.CompilerParams(
        use_tc_tiling_on_sc=False, needs_layout_passes=False
    ),
)


def kernel(responses, response_table, position_table):
    resp4 = (responses.astype(jnp.int32)
             .reshape(B // BB, BB, S // 8, 8).transpose(2, 0, 3, 1))
    raw = _sc_kernel(resp4, response_table, position_table)
    raw5 = raw.reshape(S, D // 8, B // BB, 8, BB)
    return raw5.transpose(2, 4, 0, 1, 3).reshape(B, S, D)


# speedup vs baseline: 1.0123x; 1.0123x over previous
import jax
import jax.numpy as jnp
from jax import lax
from jax.experimental import pallas as pl
from jax.experimental.pallas import tpu as pltpu
from jax.experimental.pallas import tpu_sc as plsc

N_RESP = 100000
D = 32
S = 200
B = 4096
NC = 2
NS = 16
NW = NC * NS
BB = B // NW
GROUP = 4
NGROUP = S // GROUP
NPAIR = NGROUP // 2
TILE = 8 * BB
D8STRIDE = NW * TILE
SROW = (D // 8) * D8STRIDE
OUT_WORDS = S * SROW
PITCH = BB + 1
OROWS = GROUP * 4 * 8


def _body(resp4_hbm, table_hbm, pos_hbm, out_hbm,
          idx_v, rowsA, rowsB, outA, outB, pos_v, gA, gB, sA, sB):
    wid = lax.axis_index("s") * NC + lax.axis_index("c")
    pltpu.sync_copy(pos_hbm, pos_v)
    pltpu.sync_copy(resp4_hbm.at[:, wid], idx_v)

    lane = lax.iota(jnp.int32, 16)
    rows_lo = lane
    rows_hi = lane + 16
    cols = [jnp.full((16,), u, jnp.int32) for u in range(16)]

    def fire_g(g, rows, sem):
        for sl in range(GROUP):
            s = g * GROUP + sl
            pltpu.async_copy(
                table_hbm.at[idx_v.at[s >> 3, s & 7]],
                rows.at[pl.ds(sl * BB, BB)],
                sem,
            )

    def drain_g(rows, sem):
        for sl in range(GROUP):
            pltpu.make_async_copy(
                table_hbm.at[idx_v.at[0, 0]], rows.at[pl.ds(sl * BB, BB)], sem
            ).wait()

    def fire_st(s0, outbuf, sem):
        for sl in range(GROUP):
            for d8 in range(4):
                pltpu.async_copy(
                    outbuf.at[pl.ds((sl * 4 + d8) * 8, 8), pl.ds(0, BB)],
                    out_hbm.at[pl.ds((s0 + sl) * (SROW // BB)
                                     + d8 * (D8STRIDE // BB) + wid * 8, 8), :],
                    sem,
                )

    def drain_st(outbuf, sem):
        pltpu.make_async_copy(
            outbuf.at[pl.ds(0, OROWS), pl.ds(0, BB)],
            out_hbm.at[pl.ds(0, OROWS), :], sem
        ).wait()

    def compute(s0, rows, outbuf):
        for sl in range(GROUP):
            s = s0 + sl
            p0 = pos_v[s, pl.ds(0, 16)]
            p1 = pos_v[s, pl.ds(16, 16)]

            @plsc.parallel_loop(0, BB // 4)
            def bi_body(k, sl=sl, p0=p0, p1=p1):
                dst = outbuf.at[pl.ds(sl * 32, 32)]
                lo = [rows[sl * BB + k * 4 + u, pl.ds(0, 16)] + p0
                      for u in range(4)]
                hi = [rows[sl * BB + k * 4 + u, pl.ds(16, 16)] + p1
                      for u in range(4)]
                cks = [cols[u] + k * 4 for u in range(4)]
                for u in range(4):
                    plsc.store_scatter(dst, [rows_lo, cks[u]], lo[u])
                    plsc.store_scatter(dst, [rows_hi, cks[u]], hi[u])

    fire_g(0, rowsA, gA)

    def pair_body(gp, carry):
        gA_id = 2 * gp
        s0 = gA_id * GROUP

        pl.when(gp > 0)(lambda: drain_st(outA, sA))
        fire_g(gA_id + 1, rowsB, gB)
        drain_g(rowsA, gA)
        compute(s0, rowsA, outA)
        fire_st(s0, outA, sA)

        pl.when(gp > 0)(lambda: drain_st(outB, sB))

        def _fire_next():
            fire_g(gA_id + 2, rowsA, gA)

        pl.when(gp < NPAIR - 1)(_fire_next)
        drain_g(rowsB, gB)
        compute(s0 + GROUP, rowsB, outB)
        fire_st(s0 + GROUP, outB, sB)
        return carry

    lax.fori_loop(0, NPAIR, pair_body, 0)
    drain_st(outA, sA)
    drain_st(outB, sB)


_sc_kernel = pl.kernel(
    _body,
    out_type=jax.ShapeDtypeStruct((OUT_WORDS // BB, BB), jnp.float32),
    mesh=plsc.VectorSubcoreMesh(
        core_axis_name="c", subcore_axis_name="s", num_cores=NC, num_subcores=NS
    ),
    scratch_types=[
        pltpu.VMEM((S // 8, 8, BB), jnp.int32),
        pltpu.VMEM((GROUP * BB, D), jnp.float32),
        pltpu.VMEM((GROUP * BB, D), jnp.float32),
        pltpu.VMEM((OROWS, PITCH), jnp.float32),
        pltpu.VMEM((OROWS, PITCH), jnp.float32),
        pltpu.VMEM((S, D), jnp.float32),
        pltpu.SemaphoreType.DMA,
        pltpu.SemaphoreType.DMA,
        pltpu.SemaphoreType.DMA,
        pltpu.SemaphoreType.DMA,
    ],
    compiler_params=pltpu.CompilerParams(
        use_tc_tiling_on_sc=False, needs_layout_passes=False
    ),
)


def kernel(responses, response_table, position_table):
    resp4 = (responses.astype(jnp.int32)
             .reshape(B // BB, BB, S // 8, 8).transpose(2, 0, 3, 1))
    raw = _sc_kernel(resp4, response_table, position_table)
    raw5 = raw.reshape(S, D // 8, B // BB, 8, BB)
    return raw5.transpose(2, 4, 0, 1, 3).reshape(B, S, D)


# speedup vs baseline: 1.0125x; 1.0002x over previous
import jax
import jax.numpy as jnp
from jax import lax
from jax.experimental import pallas as pl
from jax.experimental.pallas import tpu as pltpu
from jax.experimental.pallas import tpu_sc as plsc

N_RESP = 100000
D = 32
S = 200
B = 4096
NC = 2
NS = 16
NW = NC * NS
BB = B // NW
GROUP = 4
NGROUP = S // GROUP
NPAIR = NGROUP // 2
TILE = 8 * BB
D8STRIDE = NW * TILE
SROW = (D // 8) * D8STRIDE
OUT_WORDS = S * SROW
PITCH = BB + 1
OROWS = GROUP * 4 * 8


def _body(resp4_hbm, table_hbm, pos_hbm, out_hbm,
          idx_v, rowsA, rowsB, outA, outB, pos_v, gA, gB, sA, sB):
    wid = lax.axis_index("s") * NC + lax.axis_index("c")
    pltpu.sync_copy(pos_hbm, pos_v)
    pltpu.sync_copy(resp4_hbm.at[:, wid], idx_v)

    lane = lax.iota(jnp.int32, 16)
    rows_lo = lane
    rows_hi = lane + 16
    cols = [jnp.full((16,), u, jnp.int32) for u in range(16)]

    def fire_g(g, rows, sem):
        for sl in range(GROUP):
            s = g * GROUP + sl
            pltpu.async_copy(
                table_hbm.at[idx_v.at[s >> 3, s & 7]],
                rows.at[pl.ds(sl * BB, BB)],
                sem,
            )

    def drain_g(rows, sem):
        for sl in range(GROUP):
            pltpu.make_async_copy(
                table_hbm.at[idx_v.at[0, 0]], rows.at[pl.ds(sl * BB, BB)], sem
            ).wait()

    def fire_st(s0, outbuf, sem):
        for sl in range(GROUP):
            for d8 in range(4):
                pltpu.async_copy(
                    outbuf.at[pl.ds((sl * 4 + d8) * 8, 8), pl.ds(0, BB)],
                    out_hbm.at[pl.ds((s0 + sl) * (SROW // BB)
                                     + d8 * (D8STRIDE // BB) + wid * 8, 8), :],
                    sem,
                )

    def drain_st(outbuf, sem):
        pltpu.make_async_copy(
            outbuf.at[pl.ds(0, OROWS), pl.ds(0, BB)],
            out_hbm.at[pl.ds(0, OROWS), :], sem
        ).wait()

    def compute(s0, rows, outbuf):
        for sl in range(GROUP):
            s = s0 + sl
            p0 = pos_v[s, pl.ds(0, 16)]
            p1 = pos_v[s, pl.ds(16, 16)]

            @plsc.parallel_loop(0, BB // 8)
            def bi_body(k, sl=sl, p0=p0, p1=p1):
                dst = outbuf.at[pl.ds(sl * 32, 32)]
                lo = [rows[sl * BB + k * 8 + u, pl.ds(0, 16)] + p0
                      for u in range(8)]
                hi = [rows[sl * BB + k * 8 + u, pl.ds(16, 16)] + p1
                      for u in range(8)]
                cks = [cols[u] + k * 8 for u in range(8)]
                for u in range(8):
                    plsc.store_scatter(dst, [rows_lo, cks[u]], lo[u])
                    plsc.store_scatter(dst, [rows_hi, cks[u]], hi[u])

    fire_g(0, rowsA, gA)

    def pair_body(gp, carry):
        gA_id = 2 * gp
        s0 = gA_id * GROUP

        pl.when(gp > 0)(lambda: drain_st(outA, sA))
        fire_g(gA_id + 1, rowsB, gB)
        drain_g(rowsA, gA)
        compute(s0, rowsA, outA)
        fire_st(s0, outA, sA)

        pl.when(gp > 0)(lambda: drain_st(outB, sB))

        def _fire_next():
            fire_g(gA_id + 2, rowsA, gA)

        pl.when(gp < NPAIR - 1)(_fire_next)
        drain_g(rowsB, gB)
        compute(s0 + GROUP, rowsB, outB)
        fire_st(s0 + GROUP, outB, sB)
        return carry

    lax.fori_loop(0, NPAIR, pair_body, 0)
    drain_st(outA, sA)
    drain_st(outB, sB)


_sc_kernel = pl.kernel(
    _body,
    out_type=jax.ShapeDtypeStruct((OUT_WORDS // BB, BB), jnp.float32),
    mesh=plsc.VectorSubcoreMesh(
        core_axis_name="c", subcore_axis_name="s", num_cores=NC, num_subcores=NS
    ),
    scratch_types=[
        pltpu.VMEM((S // 8, 8, BB), jnp.int32),
        pltpu.VMEM((GROUP * BB, D), jnp.float32),
        pltpu.VMEM((GROUP * BB, D), jnp.float32),
        pltpu.VMEM((OROWS, PITCH), jnp.float32),
        pltpu.VMEM((OROWS, PITCH), jnp.float32),
        pltpu.VMEM((S, D), jnp.float32),
        pltpu.SemaphoreType.DMA,
        pltpu.SemaphoreType.DMA,
        pltpu.SemaphoreType.DMA,
        pltpu.SemaphoreType.DMA,
    ],
    compiler_params=pltpu.CompilerParams(
        use_tc_tiling_on_sc=False, needs_layout_passes=False
    ),
)


def kernel(responses, response_table, position_table):
    resp4 = (responses.astype(jnp.int32)
             .reshape(B // BB, BB, S // 8, 8).transpose(2, 0, 3, 1))
    raw = _sc_kernel(resp4, response_table, position_table)
    raw5 = raw.reshape(S, D // 8, B // BB, 8, BB)
    return raw5.transpose(2, 4, 0, 1, 3).reshape(B, S, D)
